# baseline (speedup 1.0000x reference)
import jax
import jax.numpy as jnp
from jax.experimental import pallas as pl
from jax.experimental.pallas import tpu as pltpu

_CHUNK = 512
_NBUF = 4
_NCHUNKS = 8192 // _CHUNK


def _dma_body(table_ref, out_ref, vbuf, in_sems, out_sems):
    ins = []
    outs = []
    for i in range(_NCHUNKS):
        b = i % _NBUF
        src = table_ref.at[pl.ds(i * _CHUNK, _CHUNK)]
        dst = out_ref.at[pl.ds(i * _CHUNK, _CHUNK)]
        ins.append(pltpu.make_async_copy(src, vbuf.at[b], in_sems.at[b]))
        outs.append(pltpu.make_async_copy(vbuf.at[b], dst, out_sems.at[b]))
    ins[0].start()
    ins[1].start()
    for i in range(_NCHUNKS):
        ins[i].wait()
        outs[i].start()
        nxt = i + 2
        if nxt < _NCHUNKS:
            if nxt >= _NBUF:
                outs[nxt - _NBUF].wait()
            ins[nxt].start()
    for i in range(_NCHUNKS - _NBUF, _NCHUNKS):
        outs[i].wait()


def kernel(x, pos_table):
    seqlen = x.shape[-1]
    embed = pos_table.shape[1]
    return pl.pallas_call(
        _dma_body,
        in_specs=[pl.BlockSpec(memory_space=pltpu.MemorySpace.HBM)],
        out_specs=pl.BlockSpec(memory_space=pltpu.MemorySpace.HBM),
        out_shape=jax.ShapeDtypeStruct((seqlen, embed), pos_table.dtype),
        scratch_shapes=[
            pltpu.VMEM((_NBUF, _CHUNK, embed), pos_table.dtype),
            pltpu.SemaphoreType.DMA((_NBUF,)),
            pltpu.SemaphoreType.DMA((_NBUF,)),
        ],
    )(pos_table)


# speedup vs baseline: 1.3034x; 1.3034x over previous
import jax
import jax.numpy as jnp
from jax.experimental import pallas as pl
from jax.experimental.pallas import tpu as pltpu

_BLK_ROWS = 3712


def _copy_body(table_ref, out_ref):
    out_ref[...] = table_ref[...]


def kernel(x, pos_table):
    seqlen = x.shape[-1]
    embed = pos_table.shape[1]
    nblk = pl.cdiv(seqlen, _BLK_ROWS)
    return pl.pallas_call(
        _copy_body,
        grid=(nblk,),
        in_specs=[pl.BlockSpec((_BLK_ROWS, embed), lambda i: (i, 0))],
        out_specs=pl.BlockSpec((_BLK_ROWS, embed), lambda i: (i, 0)),
        out_shape=jax.ShapeDtypeStruct((seqlen, embed), pos_table.dtype),
        compiler_params=pltpu.CompilerParams(
            dimension_semantics=("arbitrary",),
        ),
    )(pos_table)
